# baseline (speedup 1.0000x reference)
import jax
import jax.numpy as jnp
from jax import lax
from jax.experimental import pallas as pl
from jax.experimental.pallas import tpu as pltpu
from jax.experimental.pallas import tpu_sc as plsc

N_NODES = 10000
N_EDGES = 10000
NNZ = 320000
DF = 128
NC = 2
NS = 16
NW = NC * NS
PPW = NNZ // NW
K = 40
NCH = PPW // K
NBUF = 2
NACC = 10240
RPT = NACC // NS
CPAD = 10240
CPT = CPAD // NS

_MESH = plsc.VectorSubcoreMesh(
    core_axis_name="c", subcore_axis_name="s", num_cores=NC, num_subcores=NS)


def _sc_pass(tbl, gidx, sidx, zrows, zcnt=None, ones_h=None):
  with_counts = zcnt is not None
  outs = (jax.ShapeDtypeStruct((NC, NACC, DF), jnp.float32),)
  if with_counts:
    outs += (jax.ShapeDtypeStruct((NC, CPAD), jnp.float32),
             jax.ShapeDtypeStruct((NC, CPAD), jnp.float32))
  scratch = [
      pltpu.VMEM((PPW,), jnp.int32),
      pltpu.VMEM((PPW,), jnp.int32),
  ] + [pltpu.VMEM((K, DF), jnp.float32) for _ in range(NBUF)]
  if with_counts:
    scratch.append(pltpu.VMEM((K,), jnp.float32))
  scratch += [pltpu.VMEM_SHARED((NACC, DF), jnp.float32)]
  if with_counts:
    scratch += [pltpu.VMEM_SHARED((CPAD,), jnp.float32),
                pltpu.VMEM_SHARED((CPAD,), jnp.float32)]
  scratch += [pltpu.SemaphoreType.DMA for _ in range(2 * NBUF)]

  def body(*refs):
    if with_counts:
      (tbl_h, gidx_h, sidx_h, zrows_h, zcnt_h, ones_hb,
       acc_out, dcnt_out, bcnt_out, gidx_v, sidx_v) = refs[:11]
      bufs = refs[11:11 + NBUF]
      ones_v, acc_sh, dcnt_sh, bcnt_sh = refs[11 + NBUF:15 + NBUF]
      sems = refs[15 + NBUF:]
    else:
      (tbl_h, gidx_h, sidx_h, zrows_h, acc_out, gidx_v, sidx_v) = refs[:7]
      bufs = refs[7:7 + NBUF]
      acc_sh = refs[7 + NBUF]
      sems = refs[8 + NBUF:]
    gsems, ssems = sems[:NBUF], sems[NBUF:]
    cid = lax.axis_index("c")
    sid = lax.axis_index("s")
    wid = sid * NC + cid

    pltpu.sync_copy(gidx_h.at[wid], gidx_v)
    pltpu.sync_copy(sidx_h.at[wid], sidx_v)
    pltpu.sync_copy(zrows_h, acc_sh.at[pl.ds(sid * RPT, RPT)])
    if with_counts:
      pltpu.sync_copy(ones_hb, ones_v)
      pltpu.sync_copy(zcnt_h.at[pl.ds(sid * CPT, CPT)],
                      dcnt_sh.at[pl.ds(sid * CPT, CPT)])
      pltpu.sync_copy(zcnt_h.at[pl.ds(sid * CPT, CPT)],
                      bcnt_sh.at[pl.ds(sid * CPT, CPT)])
    plsc.subcore_barrier()

    def fire_gather(j, b):
      pltpu.async_copy(tbl_h.at[gidx_v.at[pl.ds(j * K, K)]], bufs[b], gsems[b])

    def wait_gather(b):
      pltpu.make_async_copy(
          tbl_h.at[gidx_v.at[pl.ds(0, K)]], bufs[b], gsems[b]).wait()

    def fire_scatter(j, b):
      pltpu.async_copy(
          bufs[b], acc_sh.at[sidx_v.at[pl.ds(j * K, K)]], ssems[b], add=True)

    def wait_scatter(b):
      pltpu.make_async_copy(
          bufs[b], acc_sh.at[sidx_v.at[pl.ds(0, K)]], ssems[b]).wait()

    for b in range(NBUF):
      fire_gather(b, b)

    def step(t, carry):
      j = t * NBUF
      for b in range(NBUF):
        wait_gather(b)
        fire_scatter(j + b, b)
        if with_counts:
          pltpu.sync_copy(
              ones_v, dcnt_sh.at[gidx_v.at[pl.ds((j + b) * K, K)]], add=True)
          pltpu.sync_copy(
              ones_v, bcnt_sh.at[sidx_v.at[pl.ds((j + b) * K, K)]], add=True)
      for b in range(NBUF):
        wait_scatter(b)

        @pl.when(j + b + NBUF < NCH)
        def _():
          fire_gather(j + b + NBUF, b)

      return carry

    lax.fori_loop(0, NCH // NBUF, step, 0)
    for r in range(NCH % NBUF):
      j = (NCH // NBUF) * NBUF + r
      wait_gather(r)
      fire_scatter(j, r)
      if with_counts:
        pltpu.sync_copy(
            ones_v, dcnt_sh.at[gidx_v.at[pl.ds(j * K, K)]], add=True)
        pltpu.sync_copy(
            ones_v, bcnt_sh.at[sidx_v.at[pl.ds(j * K, K)]], add=True)
    for r in range(NCH % NBUF):
      wait_scatter(r)
    plsc.subcore_barrier()
    pltpu.sync_copy(acc_sh.at[pl.ds(sid * RPT, RPT)],
                    acc_out.at[cid, pl.ds(sid * RPT, RPT)])
    if with_counts:
      pltpu.sync_copy(dcnt_sh.at[pl.ds(sid * CPT, CPT)],
                      dcnt_out.at[cid, pl.ds(sid * CPT, CPT)])
      pltpu.sync_copy(bcnt_sh.at[pl.ds(sid * CPT, CPT)],
                      bcnt_out.at[cid, pl.ds(sid * CPT, CPT)])

  fn = pl.kernel(body, out_type=outs, mesh=_MESH, scratch_types=tuple(scratch))
  if with_counts:
    return fn(tbl, gidx, sidx, zrows, zcnt, ones_h)
  return fn(tbl, gidx, sidx, zrows)


_BM = 1024
_BMF = 1000


def _tc_combine_scale(part, cnt):
  def c1(p_ref, c_ref, o_ref):
    s = p_ref[0] + p_ref[1]
    c = c_ref[0] + c_ref[1]
    cinv = jnp.where(c == 0, 0.0, 1.0 / c)
    o_ref[...] = s * cinv

  return pl.pallas_call(
      c1,
      grid=(NACC // _BM,),
      in_specs=[pl.BlockSpec((NC, _BM, DF), lambda i: (0, i, 0)),
                pl.BlockSpec((NC, _BM, 1), lambda i: (0, i, 0))],
      out_specs=pl.BlockSpec((_BM, DF), lambda i: (i, 0)),
      out_shape=jax.ShapeDtypeStruct((NACC, DF), jnp.float32),
  )(part, cnt.reshape(NC, CPAD, 1))


def _tc_matmul(X, W):
  def mm(x_ref, w_ref, o_ref):
    o_ref[...] = jnp.dot(x_ref[...], w_ref[...],
                         preferred_element_type=jnp.float32)

  return pl.pallas_call(
      mm,
      grid=(N_NODES // _BMF,),
      in_specs=[pl.BlockSpec((_BMF, DF), lambda i: (i, 0)),
                pl.BlockSpec((DF, DF), lambda i: (0, 0))],
      out_specs=pl.BlockSpec((_BMF, DF), lambda i: (i, 0)),
      out_shape=jax.ShapeDtypeStruct((N_NODES, DF), jnp.float32),
  )(X, W)


def _tc_finalize(part, cnt, b2d):
  def c2(p_ref, c_ref, b_ref, o_ref):
    s = p_ref[0] + p_ref[1]
    c = c_ref[0] + c_ref[1]
    cinv = jnp.where(c == 0, 0.0, 1.0 / c)
    o = s * cinv + b_ref[...]
    o_ref[...] = jnp.where(o >= 0, o, 0.01 * o)

  return pl.pallas_call(
      c2,
      grid=(N_NODES // _BMF,),
      in_specs=[pl.BlockSpec((NC, _BMF, DF), lambda i: (0, i, 0)),
                pl.BlockSpec((NC, _BMF, 1), lambda i: (0, i, 0)),
                pl.BlockSpec((1, DF), lambda i: (0, 0))],
      out_specs=pl.BlockSpec((_BMF, DF), lambda i: (i, 0)),
      out_shape=jax.ShapeDtypeStruct((N_NODES, DF), jnp.float32),
  )(part, cnt.reshape(NC, CPAD, 1), b2d)


def kernel(X, A, W, b):
  nidx = A[0].astype(jnp.int32).reshape(NW, PPW)
  eidx = A[1].astype(jnp.int32).reshape(NW, PPW)
  zrows = jnp.zeros((RPT, DF), jnp.float32)
  zcnt = jnp.zeros((CPAD,), jnp.float32)
  ones_h = jnp.ones((K,), jnp.float32)

  x = _tc_matmul(X, W)
  hpart, dcnt, bcnt = _sc_pass(x, nidx, eidx, zrows, zcnt, ones_h)
  hedge = _tc_combine_scale(hpart, bcnt)
  opart, = _sc_pass(hedge, eidx, nidx, zrows)
  return _tc_finalize(opart, dcnt, b.reshape(1, DF))


# speedup vs baseline: 1.1685x; 1.1685x over previous
import jax
import jax.numpy as jnp
from jax import lax
from jax.experimental import pallas as pl
from jax.experimental.pallas import tpu as pltpu
from jax.experimental.pallas import tpu_sc as plsc

N_NODES = 10000
N_EDGES = 10000
NNZ = 320000
DF = 128
NC = 2
NS = 16
NW = NC * NS
PPW = NNZ // NW
K = 80
NCH = PPW // K
NBUF = 2
NACC = 10240
RPT = NACC // NS
CPAD = 10240
CPT = CPAD // NS

_MESH = plsc.VectorSubcoreMesh(
    core_axis_name="c", subcore_axis_name="s", num_cores=NC, num_subcores=NS)


def _sc_pass(tbl, gidx, sidx, zrows, zcnt=None, ones_h=None):
  with_counts = zcnt is not None
  outs = (jax.ShapeDtypeStruct((NC, NACC, DF), jnp.float32),)
  if with_counts:
    outs += (jax.ShapeDtypeStruct((NC, CPAD), jnp.float32),
             jax.ShapeDtypeStruct((NC, CPAD), jnp.float32))
  scratch = [
      pltpu.VMEM((PPW,), jnp.int32),
      pltpu.VMEM((PPW,), jnp.int32),
  ] + [pltpu.VMEM((K, DF), jnp.float32) for _ in range(NBUF)]
  if with_counts:
    scratch.append(pltpu.VMEM((K,), jnp.float32))
  scratch += [pltpu.VMEM_SHARED((NACC, DF), jnp.float32)]
  if with_counts:
    scratch += [pltpu.VMEM_SHARED((CPAD,), jnp.float32),
                pltpu.VMEM_SHARED((CPAD,), jnp.float32)]
  scratch += [pltpu.SemaphoreType.DMA for _ in range(2 * NBUF)]

  def body(*refs):
    if with_counts:
      (tbl_h, gidx_h, sidx_h, zrows_h, zcnt_h, ones_hb,
       acc_out, dcnt_out, bcnt_out, gidx_v, sidx_v) = refs[:11]
      bufs = refs[11:11 + NBUF]
      ones_v, acc_sh, dcnt_sh, bcnt_sh = refs[11 + NBUF:15 + NBUF]
      sems = refs[15 + NBUF:]
    else:
      (tbl_h, gidx_h, sidx_h, zrows_h, acc_out, gidx_v, sidx_v) = refs[:7]
      bufs = refs[7:7 + NBUF]
      acc_sh = refs[7 + NBUF]
      sems = refs[8 + NBUF:]
    gsems, ssems = sems[:NBUF], sems[NBUF:]
    cid = lax.axis_index("c")
    sid = lax.axis_index("s")
    wid = sid * NC + cid

    pltpu.sync_copy(gidx_h.at[wid], gidx_v)
    pltpu.sync_copy(sidx_h.at[wid], sidx_v)
    pltpu.sync_copy(zrows_h, acc_sh.at[pl.ds(sid * RPT, RPT)])
    if with_counts:
      pltpu.sync_copy(ones_hb, ones_v)
      pltpu.sync_copy(zcnt_h.at[pl.ds(sid * CPT, CPT)],
                      dcnt_sh.at[pl.ds(sid * CPT, CPT)])
      pltpu.sync_copy(zcnt_h.at[pl.ds(sid * CPT, CPT)],
                      bcnt_sh.at[pl.ds(sid * CPT, CPT)])
    plsc.subcore_barrier()

    def fire_gather(j, b):
      pltpu.async_copy(tbl_h.at[gidx_v.at[pl.ds(j * K, K)]], bufs[b], gsems[b])

    def wait_gather(b):
      pltpu.make_async_copy(
          tbl_h.at[gidx_v.at[pl.ds(0, K)]], bufs[b], gsems[b]).wait()

    def fire_scatter(j, b):
      pltpu.async_copy(
          bufs[b], acc_sh.at[sidx_v.at[pl.ds(j * K, K)]], ssems[b], add=True)

    def wait_scatter(b):
      pltpu.make_async_copy(
          bufs[b], acc_sh.at[sidx_v.at[pl.ds(0, K)]], ssems[b]).wait()

    for b in range(NBUF):
      fire_gather(b, b)

    def step(t, carry):
      j = t * NBUF
      for b in range(NBUF):
        wait_gather(b)
        fire_scatter(j + b, b)
        if with_counts:
          pltpu.sync_copy(
              ones_v, dcnt_sh.at[gidx_v.at[pl.ds((j + b) * K, K)]], add=True)
          pltpu.sync_copy(
              ones_v, bcnt_sh.at[sidx_v.at[pl.ds((j + b) * K, K)]], add=True)
      for b in range(NBUF):
        wait_scatter(b)

        @pl.when(j + b + NBUF < NCH)
        def _():
          fire_gather(j + b + NBUF, b)

      return carry

    lax.fori_loop(0, NCH // NBUF, step, 0)
    for r in range(NCH % NBUF):
      j = (NCH // NBUF) * NBUF + r
      wait_gather(r)
      fire_scatter(j, r)
      if with_counts:
        pltpu.sync_copy(
            ones_v, dcnt_sh.at[gidx_v.at[pl.ds(j * K, K)]], add=True)
        pltpu.sync_copy(
            ones_v, bcnt_sh.at[sidx_v.at[pl.ds(j * K, K)]], add=True)
    for r in range(NCH % NBUF):
      wait_scatter(r)
    plsc.subcore_barrier()
    pltpu.sync_copy(acc_sh.at[pl.ds(sid * RPT, RPT)],
                    acc_out.at[cid, pl.ds(sid * RPT, RPT)])
    if with_counts:
      pltpu.sync_copy(dcnt_sh.at[pl.ds(sid * CPT, CPT)],
                      dcnt_out.at[cid, pl.ds(sid * CPT, CPT)])
      pltpu.sync_copy(bcnt_sh.at[pl.ds(sid * CPT, CPT)],
                      bcnt_out.at[cid, pl.ds(sid * CPT, CPT)])

  fn = pl.kernel(body, out_type=outs, mesh=_MESH, scratch_types=tuple(scratch))
  if with_counts:
    return fn(tbl, gidx, sidx, zrows, zcnt, ones_h)
  return fn(tbl, gidx, sidx, zrows)


_BM = 1024
_BMF = 1000


def _tc_combine_scale(part, cnt):
  def c1(p_ref, c_ref, o_ref):
    s = p_ref[0] + p_ref[1]
    c = c_ref[0] + c_ref[1]
    cinv = jnp.where(c == 0, 0.0, 1.0 / c)
    o_ref[...] = s * cinv

  return pl.pallas_call(
      c1,
      grid=(NACC // _BM,),
      in_specs=[pl.BlockSpec((NC, _BM, DF), lambda i: (0, i, 0)),
                pl.BlockSpec((NC, _BM, 1), lambda i: (0, i, 0))],
      out_specs=pl.BlockSpec((_BM, DF), lambda i: (i, 0)),
      out_shape=jax.ShapeDtypeStruct((NACC, DF), jnp.float32),
  )(part, cnt.reshape(NC, CPAD, 1))


def _tc_finalize(part, cnt, W, b2d):
  def c2(p_ref, c_ref, w_ref, b_ref, o_ref):
    s = p_ref[0] + p_ref[1]
    c = c_ref[0] + c_ref[1]
    cinv = jnp.where(c == 0, 0.0, 1.0 / c)
    o = jnp.dot(s * cinv, w_ref[...],
                preferred_element_type=jnp.float32) + b_ref[...]
    o_ref[...] = jnp.where(o >= 0, o, 0.01 * o)

  return pl.pallas_call(
      c2,
      grid=(N_NODES // _BMF,),
      in_specs=[pl.BlockSpec((NC, _BMF, DF), lambda i: (0, i, 0)),
                pl.BlockSpec((NC, _BMF, 1), lambda i: (0, i, 0)),
                pl.BlockSpec((DF, DF), lambda i: (0, 0)),
                pl.BlockSpec((1, DF), lambda i: (0, 0))],
      out_specs=pl.BlockSpec((_BMF, DF), lambda i: (i, 0)),
      out_shape=jax.ShapeDtypeStruct((N_NODES, DF), jnp.float32),
  )(part, cnt.reshape(NC, CPAD, 1), W, b2d)


def kernel(X, A, W, b):
  nidx = A[0].astype(jnp.int32).reshape(NW, PPW)
  eidx = A[1].astype(jnp.int32).reshape(NW, PPW)
  zrows = jnp.zeros((RPT, DF), jnp.float32)
  zcnt = jnp.zeros((CPAD,), jnp.float32)
  ones_h = jnp.ones((K,), jnp.float32)

  hpart, dcnt, bcnt = _sc_pass(X, nidx, eidx, zrows, zcnt, ones_h)
  hedge = _tc_combine_scale(hpart, bcnt)
  opart, = _sc_pass(hedge, eidx, nidx, zrows)
  return _tc_finalize(opart, dcnt, W, b.reshape(1, DF))


# speedup vs baseline: 1.1744x; 1.0050x over previous
import jax
import jax.numpy as jnp
from jax import lax
from jax.experimental import pallas as pl
from jax.experimental.pallas import tpu as pltpu
from jax.experimental.pallas import tpu_sc as plsc

N_NODES = 10000
N_EDGES = 10000
NNZ = 320000
DF = 128
NC = 2
NS = 16
NW = NC * NS
PPW = NNZ // NW
K = 80
NCH = PPW // K
NBUF = 2
NACC = 10240
RPT = NACC // NS
CPAD = 10240
CPT = CPAD // NS

_MESH = plsc.VectorSubcoreMesh(
    core_axis_name="c", subcore_axis_name="s", num_cores=NC, num_subcores=NS)


def _sc_pass(tbl, gidx, sidx, zrows, zcnt=None, ones_h=None):
  with_counts = zcnt is not None
  outs = (jax.ShapeDtypeStruct((NC, NACC, DF), jnp.float32),)
  if with_counts:
    outs += (jax.ShapeDtypeStruct((NC, CPAD), jnp.float32),
             jax.ShapeDtypeStruct((NC, CPAD), jnp.float32))
  scratch = [
      pltpu.VMEM((PPW,), jnp.int32),
      pltpu.VMEM((PPW,), jnp.int32),
  ] + [pltpu.VMEM((K, DF), tbl.dtype) for _ in range(NBUF)]
  if with_counts:
    scratch.append(pltpu.VMEM((K,), jnp.float32))
  scratch += [pltpu.VMEM_SHARED((NACC, DF), jnp.float32)]
  if with_counts:
    scratch += [pltpu.VMEM_SHARED((CPAD,), jnp.float32),
                pltpu.VMEM_SHARED((CPAD,), jnp.float32)]
  scratch += [pltpu.SemaphoreType.DMA for _ in range(2 * NBUF)]

  def body(*refs):
    if with_counts:
      (tbl_h, gidx_h, sidx_h, zrows_h, zcnt_h, ones_hb,
       acc_out, dcnt_out, bcnt_out, gidx_v, sidx_v) = refs[:11]
      bufs = refs[11:11 + NBUF]
      ones_v, acc_sh, dcnt_sh, bcnt_sh = refs[11 + NBUF:15 + NBUF]
      sems = refs[15 + NBUF:]
    else:
      (tbl_h, gidx_h, sidx_h, zrows_h, acc_out, gidx_v, sidx_v) = refs[:7]
      bufs = refs[7:7 + NBUF]
      acc_sh = refs[7 + NBUF]
      sems = refs[8 + NBUF:]
    gsems, ssems = sems[:NBUF], sems[NBUF:]
    cid = lax.axis_index("c")
    sid = lax.axis_index("s")
    wid = sid * NC + cid

    pltpu.sync_copy(gidx_h.at[wid], gidx_v)
    pltpu.sync_copy(sidx_h.at[wid], sidx_v)
    pltpu.sync_copy(zrows_h, bufs[0])
    for q in range(RPT // K):
      pltpu.sync_copy(bufs[0], acc_sh.at[pl.ds(sid * RPT + q * K, K)])
    if with_counts:
      pltpu.sync_copy(ones_hb, ones_v)
      pltpu.sync_copy(zcnt_h.at[pl.ds(sid * CPT, CPT)],
                      dcnt_sh.at[pl.ds(sid * CPT, CPT)])
      pltpu.sync_copy(zcnt_h.at[pl.ds(sid * CPT, CPT)],
                      bcnt_sh.at[pl.ds(sid * CPT, CPT)])
    plsc.subcore_barrier()

    def fire_gather(j, b):
      pltpu.async_copy(tbl_h.at[gidx_v.at[pl.ds(j * K, K)]], bufs[b], gsems[b])

    def wait_gather(b):
      pltpu.make_async_copy(
          tbl_h.at[gidx_v.at[pl.ds(0, K)]], bufs[b], gsems[b]).wait()

    def fire_scatter(j, b):
      pltpu.async_copy(
          bufs[b], acc_sh.at[sidx_v.at[pl.ds(j * K, K)]], ssems[b], add=True)

    def wait_scatter(b):
      pltpu.make_async_copy(
          bufs[b], acc_sh.at[sidx_v.at[pl.ds(0, K)]], ssems[b]).wait()

    for b in range(NBUF):
      fire_gather(b, b)

    def step(t, carry):
      j = t * NBUF
      for b in range(NBUF):
        wait_gather(b)
        fire_scatter(j + b, b)
        if with_counts:
          pltpu.sync_copy(
              ones_v, dcnt_sh.at[gidx_v.at[pl.ds((j + b) * K, K)]], add=True)
          pltpu.sync_copy(
              ones_v, bcnt_sh.at[sidx_v.at[pl.ds((j + b) * K, K)]], add=True)
      for b in range(NBUF):
        wait_scatter(b)

        @pl.when(j + b + NBUF < NCH)
        def _():
          fire_gather(j + b + NBUF, b)

      return carry

    lax.fori_loop(0, NCH // NBUF, step, 0)
    for r in range(NCH % NBUF):
      j = (NCH // NBUF) * NBUF + r
      wait_gather(r)
      fire_scatter(j, r)
      if with_counts:
        pltpu.sync_copy(
            ones_v, dcnt_sh.at[gidx_v.at[pl.ds(j * K, K)]], add=True)
        pltpu.sync_copy(
            ones_v, bcnt_sh.at[sidx_v.at[pl.ds(j * K, K)]], add=True)
    for r in range(NCH % NBUF):
      wait_scatter(r)
    plsc.subcore_barrier()
    pltpu.sync_copy(acc_sh.at[pl.ds(sid * RPT, RPT)],
                    acc_out.at[cid, pl.ds(sid * RPT, RPT)])
    if with_counts:
      pltpu.sync_copy(dcnt_sh.at[pl.ds(sid * CPT, CPT)],
                      dcnt_out.at[cid, pl.ds(sid * CPT, CPT)])
      pltpu.sync_copy(bcnt_sh.at[pl.ds(sid * CPT, CPT)],
                      bcnt_out.at[cid, pl.ds(sid * CPT, CPT)])

  fn = pl.kernel(body, out_type=outs, mesh=_MESH, scratch_types=tuple(scratch))
  if with_counts:
    return fn(tbl, gidx, sidx, zrows, zcnt, ones_h)
  return fn(tbl, gidx, sidx, zrows)


_BM = 1024
_BMF = 1000


def _tc_combine_scale(part, cnt):
  def c1(p_ref, c_ref, o_ref):
    s = p_ref[0] + p_ref[1]
    c = c_ref[0] + c_ref[1]
    cinv = jnp.where(c == 0, 0.0, 1.0 / c)
    o_ref[...] = s * cinv

  return pl.pallas_call(
      c1,
      grid=(NACC // _BM,),
      in_specs=[pl.BlockSpec((NC, _BM, DF), lambda i: (0, i, 0)),
                pl.BlockSpec((NC, _BM, 1), lambda i: (0, i, 0))],
      out_specs=pl.BlockSpec((_BM, DF), lambda i: (i, 0)),
      out_shape=jax.ShapeDtypeStruct((NACC, DF), jnp.float32),
  )(part, cnt.reshape(NC, CPAD, 1))


def _tc_finalize(part, cnt, W, b2d):
  def c2(p_ref, c_ref, w_ref, b_ref, o_ref):
    s = p_ref[0] + p_ref[1]
    c = c_ref[0] + c_ref[1]
    cinv = jnp.where(c == 0, 0.0, 1.0 / c)
    o = jnp.dot(s * cinv, w_ref[...],
                preferred_element_type=jnp.float32) + b_ref[...]
    o_ref[...] = jnp.where(o >= 0, o, 0.01 * o)

  return pl.pallas_call(
      c2,
      grid=(N_NODES // _BMF,),
      in_specs=[pl.BlockSpec((NC, _BMF, DF), lambda i: (0, i, 0)),
                pl.BlockSpec((NC, _BMF, 1), lambda i: (0, i, 0)),
                pl.BlockSpec((DF, DF), lambda i: (0, 0)),
                pl.BlockSpec((1, DF), lambda i: (0, 0))],
      out_specs=pl.BlockSpec((_BMF, DF), lambda i: (i, 0)),
      out_shape=jax.ShapeDtypeStruct((N_NODES, DF), jnp.float32),
  )(part, cnt.reshape(NC, CPAD, 1), W, b2d)


def kernel(X, A, W, b):
  nidx = A[0].astype(jnp.int32).reshape(NW, PPW)
  eidx = A[1].astype(jnp.int32).reshape(NW, PPW)
  zrows = jnp.zeros((K, DF), jnp.float32)
  zcnt = jnp.zeros((CPAD,), jnp.float32)
  ones_h = jnp.ones((K,), jnp.float32)

  hpart, dcnt, bcnt = _sc_pass(X, nidx, eidx, zrows, zcnt, ones_h)
  hedge = _tc_combine_scale(hpart, bcnt)
  opart, = _sc_pass(hedge, eidx, nidx, zrows)
  return _tc_finalize(opart, dcnt, W, b.reshape(1, DF))


# speedup vs baseline: 1.1853x; 1.0093x over previous
import jax
import jax.numpy as jnp
from jax import lax
from jax.experimental import pallas as pl
from jax.experimental.pallas import tpu as pltpu
from jax.experimental.pallas import tpu_sc as plsc

N_NODES = 10000
N_EDGES = 10000
NNZ = 320000
DF = 128
NC = 2
NS = 16
NW = NC * NS
PPW = NNZ // NW
K = 80
NCH = PPW // K
NBUF = 2
NACC = 10240
RPT = NACC // NS
CPAD = 10240
CPT = CPAD // NS

_MESH = plsc.VectorSubcoreMesh(
    core_axis_name="c", subcore_axis_name="s", num_cores=NC, num_subcores=NS)


def _sc_pass(tbl, gidx, sidx, zrows, zcnt=None, ones_h=None):
  with_counts = zcnt is not None
  outs = (jax.ShapeDtypeStruct((NC, NACC, DF), jnp.float32),)
  if with_counts:
    outs += (jax.ShapeDtypeStruct((NC, CPAD), jnp.float32),
             jax.ShapeDtypeStruct((NC, CPAD), jnp.float32))
  scratch = [
      pltpu.VMEM((PPW,), jnp.int32),
      pltpu.VMEM((PPW,), jnp.int32),
  ] + [pltpu.VMEM((K, DF), tbl.dtype) for _ in range(NBUF)]
  if with_counts:
    scratch.append(pltpu.VMEM((K,), jnp.float32))
  scratch += [pltpu.VMEM_SHARED((NACC, DF), jnp.float32)]
  if with_counts:
    scratch += [pltpu.VMEM_SHARED((CPAD,), jnp.float32),
                pltpu.VMEM_SHARED((CPAD,), jnp.float32)]
  scratch += [pltpu.SemaphoreType.DMA for _ in range(2 * NBUF)]

  def body(*refs):
    if with_counts:
      (tbl_h, gidx_h, sidx_h, zrows_h, zcnt_h, ones_hb,
       acc_out, dcnt_out, bcnt_out, gidx_v, sidx_v) = refs[:11]
      bufs = refs[11:11 + NBUF]
      ones_v, acc_sh, dcnt_sh, bcnt_sh = refs[11 + NBUF:15 + NBUF]
      sems = refs[15 + NBUF:]
    else:
      (tbl_h, gidx_h, sidx_h, zrows_h, acc_out, gidx_v, sidx_v) = refs[:7]
      bufs = refs[7:7 + NBUF]
      acc_sh = refs[7 + NBUF]
      sems = refs[8 + NBUF:]
    gsems, ssems = sems[:NBUF], sems[NBUF:]
    cid = lax.axis_index("c")
    sid = lax.axis_index("s")
    wid = sid * NC + cid

    pltpu.sync_copy(gidx_h.at[wid], gidx_v)
    pltpu.sync_copy(sidx_h.at[wid], sidx_v)
    pltpu.sync_copy(zrows_h, bufs[0])
    for q in range(RPT // K):
      pltpu.sync_copy(bufs[0], acc_sh.at[pl.ds(sid * RPT + q * K, K)])
    if with_counts:
      pltpu.sync_copy(ones_hb, ones_v)
      pltpu.sync_copy(zcnt_h.at[pl.ds(sid * CPT, CPT)],
                      dcnt_sh.at[pl.ds(sid * CPT, CPT)])
      pltpu.sync_copy(zcnt_h.at[pl.ds(sid * CPT, CPT)],
                      bcnt_sh.at[pl.ds(sid * CPT, CPT)])
    plsc.subcore_barrier()

    def fire_gather(j, b):
      pltpu.async_copy(tbl_h.at[gidx_v.at[pl.ds(j * K, K)]], bufs[b], gsems[b])

    def wait_gather(b):
      pltpu.make_async_copy(
          tbl_h.at[gidx_v.at[pl.ds(0, K)]], bufs[b], gsems[b]).wait()

    def fire_scatter(j, b):
      pltpu.async_copy(
          bufs[b], acc_sh.at[sidx_v.at[pl.ds(j * K, K)]], ssems[b], add=True)

    def wait_scatter(b):
      pltpu.make_async_copy(
          bufs[b], acc_sh.at[sidx_v.at[pl.ds(0, K)]], ssems[b]).wait()

    for b in range(NBUF):
      fire_gather(b, b)

    def step(t, carry):
      j = t * NBUF
      for b in range(NBUF):
        wait_gather(b)
        fire_scatter(j + b, b)
        if with_counts:
          pltpu.sync_copy(
              ones_v, dcnt_sh.at[gidx_v.at[pl.ds((j + b) * K, K)]], add=True)
          pltpu.sync_copy(
              ones_v, bcnt_sh.at[sidx_v.at[pl.ds((j + b) * K, K)]], add=True)
      for b in range(NBUF):
        wait_scatter(b)

        @pl.when(j + b + NBUF < NCH)
        def _():
          fire_gather(j + b + NBUF, b)

      return carry

    lax.fori_loop(0, NCH // NBUF, step, 0)
    for r in range(NCH % NBUF):
      j = (NCH // NBUF) * NBUF + r
      wait_gather(r)
      fire_scatter(j, r)
      if with_counts:
        pltpu.sync_copy(
            ones_v, dcnt_sh.at[gidx_v.at[pl.ds(j * K, K)]], add=True)
        pltpu.sync_copy(
            ones_v, bcnt_sh.at[sidx_v.at[pl.ds(j * K, K)]], add=True)
    for r in range(NCH % NBUF):
      wait_scatter(r)
    plsc.subcore_barrier()
    pltpu.sync_copy(acc_sh.at[pl.ds(sid * RPT, RPT)],
                    acc_out.at[cid, pl.ds(sid * RPT, RPT)])
    if with_counts:
      pltpu.sync_copy(dcnt_sh.at[pl.ds(sid * CPT, CPT)],
                      dcnt_out.at[cid, pl.ds(sid * CPT, CPT)])
      pltpu.sync_copy(bcnt_sh.at[pl.ds(sid * CPT, CPT)],
                      bcnt_out.at[cid, pl.ds(sid * CPT, CPT)])

  fn = pl.kernel(body, out_type=outs, mesh=_MESH, scratch_types=tuple(scratch))
  if with_counts:
    return fn(tbl, gidx, sidx, zrows, zcnt, ones_h)
  return fn(tbl, gidx, sidx, zrows)


_BM = 2048
_BMF = 2000


def _tc_combine_scale(part, cnt):
  def c1(p_ref, c_ref, o_ref):
    s = p_ref[0] + p_ref[1]
    c = c_ref[0] + c_ref[1]
    cinv = jnp.where(c == 0, 0.0, 1.0 / c)
    o_ref[...] = s * cinv

  return pl.pallas_call(
      c1,
      grid=(NACC // _BM,),
      in_specs=[pl.BlockSpec((NC, _BM, DF), lambda i: (0, i, 0)),
                pl.BlockSpec((NC, _BM, 1), lambda i: (0, i, 0))],
      out_specs=pl.BlockSpec((_BM, DF), lambda i: (i, 0)),
      out_shape=jax.ShapeDtypeStruct((NACC, DF), jnp.float32),
  )(part, cnt.reshape(NC, CPAD, 1))


def _tc_finalize(part, cnt, W, b2d):
  def c2(p_ref, c_ref, w_ref, b_ref, o_ref):
    s = p_ref[0] + p_ref[1]
    c = c_ref[0] + c_ref[1]
    cinv = jnp.where(c == 0, 0.0, 1.0 / c)
    o = jnp.dot(s * cinv, w_ref[...],
                preferred_element_type=jnp.float32) + b_ref[...]
    o_ref[...] = jnp.where(o >= 0, o, 0.01 * o)

  return pl.pallas_call(
      c2,
      grid=(N_NODES // _BMF,),
      in_specs=[pl.BlockSpec((NC, _BMF, DF), lambda i: (0, i, 0)),
                pl.BlockSpec((NC, _BMF, 1), lambda i: (0, i, 0)),
                pl.BlockSpec((DF, DF), lambda i: (0, 0)),
                pl.BlockSpec((1, DF), lambda i: (0, 0))],
      out_specs=pl.BlockSpec((_BMF, DF), lambda i: (i, 0)),
      out_shape=jax.ShapeDtypeStruct((N_NODES, DF), jnp.float32),
  )(part, cnt.reshape(NC, CPAD, 1), W, b2d)


def kernel(X, A, W, b):
  nidx = A[0].astype(jnp.int32).reshape(NW, PPW)
  eidx = A[1].astype(jnp.int32).reshape(NW, PPW)
  zrows = jnp.zeros((K, DF), jnp.float32)
  zcnt = jnp.zeros((CPAD,), jnp.float32)
  ones_h = jnp.ones((K,), jnp.float32)

  hpart, dcnt, bcnt = _sc_pass(X, nidx, eidx, zrows, zcnt, ones_h)
  hedge = _tc_combine_scale(hpart, bcnt)
  opart, = _sc_pass(hedge, eidx, nidx, zrows)
  return _tc_finalize(opart, dcnt, W, b.reshape(1, DF))
